# Initial kernel scaffold; baseline (speedup 1.0000x reference)
import jax
import jax.numpy as jnp
from jax.experimental import pallas as pl


def kernel(points, point_features, boxes3d):
    raise NotImplementedError("write your pallas kernel here")


# speedup vs baseline: 12.9127x; 12.9127x over previous
import functools

import jax
import jax.numpy as jnp
from jax import lax
from jax.experimental import pallas as pl
from jax.experimental.pallas import tpu as pltpu
from jax.experimental.pallas import tpu_sc as plsc

_B, _N, _C, _M = 8, 16384, 16, 128
_S = 512
_F = 3 + _C
_FP = 32
_NW = 32
_NBOX = (_B * _M) // _NW
_R = _B * _N
_ZROW = _R
_L = 16
_NCHUNK = _N // _L
_IDXCAP = _S + _L


def _sc_body(px_hbm, py_hbm, pz_hbm, feats_hbm, boxes_hbm,
             pooled_hbm, empty_hbm,
             px_v, py_v, pz_v, boxbuf, idxbuf, fidx, rows_v, emptybuf, sem):
    wid = lax.axis_index("s") * 2 + lax.axis_index("c")
    gbase = wid * _NBOX
    b = gbase // _M
    pltpu.sync_copy(px_hbm.at[pl.ds(b * _N, _N)], px_v)
    pltpu.sync_copy(py_hbm.at[pl.ds(b * _N, _N)], py_v)
    pltpu.sync_copy(pz_hbm.at[pl.ds(b * _N, _N)], pz_v)
    pltpu.sync_copy(boxes_hbm.at[pl.ds(gbase * 8, _NBOX * 8)], boxbuf)
    lanes = lax.iota(jnp.int32, _L)
    lane0 = lanes == 0
    badd = b * _N

    def box_body(i, _):
        def param(p):
            return plsc.load_gather(
                boxbuf, [jnp.full((_L,), i * 8 + p, jnp.int32)])

        cx, cy, czc, hx, hy, hz, cosa, sina = [param(p) for p in range(8)]

        def chunk(k, ptr):
            base = k * _L
            pxv = px_v[pl.ds(base, _L)]
            pyv = py_v[pl.ds(base, _L)]
            pzv = pz_v[pl.ds(base, _L)]
            dxp = pxv - cx
            dyp = pyv - cy
            lx = dxp * cosa + dyp * sina
            ly = dyp * cosa - dxp * sina
            m = ((jnp.abs(pzv - czc) <= hz)
                 & (jnp.abs(lx) <= hx)
                 & (jnp.abs(ly) <= hy))
            tgt = ptr + plsc.cumsum(m.astype(jnp.int32)) - 1
            tgt = jnp.clip(tgt, 0, _IDXCAP - 1)
            plsc.store_scatter(idxbuf, [tgt], base + lanes, mask=m)
            return ptr + plsc.all_reduce_population_count(m)

        ptr = lax.fori_loop(0, _NCHUNK, chunk, jnp.zeros((_L,), jnp.int32))
        cnt = jnp.minimum(jnp.max(ptr), _S)
        cnt_v = jnp.full((_L,), cnt, jnp.int32)
        empty_v = cnt_v == 0
        cnt_safe = jnp.maximum(cnt_v, 1)

        def build(jj, _):
            jv = jj * _L + lanes
            w = lax.rem(jv, cnt_safe)
            src = plsc.load_gather(idxbuf, [w]) + badd
            fidx[pl.ds(jj * _L, _L)] = jnp.where(empty_v, _ZROW, src)
            return 0

        lax.fori_loop(0, _S // _L, build, 0)

        copies = [
            pltpu.async_copy(
                feats_hbm.at[fidx.at[pl.ds(kk * 128, 128)]],
                rows_v.at[pl.ds(kk * 128, 128)], sem)
            for kk in range(_S // 128)
        ]
        for cpy in copies:
            cpy.wait()
        pltpu.sync_copy(rows_v, pooled_hbm.at[pl.ds((gbase + i) * _S, _S)])
        plsc.store_scatter(emptybuf, [jnp.full((_L,), i, jnp.int32)],
                           empty_v.astype(jnp.int32), mask=lane0)
        return 0

    lax.fori_loop(0, _NBOX, box_body, 0)
    pltpu.sync_copy(emptybuf, empty_hbm.at[pl.ds(gbase, _NBOX)])


def kernel(points, point_features, boxes3d):
    B, N, _ = points.shape
    M = boxes3d.shape[1]
    px = points[:, :, 0].reshape(-1)
    py = points[:, :, 1].reshape(-1)
    pz = points[:, :, 2].reshape(-1)
    feats = jnp.concatenate([
        points, point_features,
        jnp.zeros((B, N, _FP - _F), jnp.float32)], axis=-1)
    feats = feats.reshape(B * N, _FP)
    feats = jnp.concatenate(
        [feats, jnp.zeros((_L, _FP), jnp.float32)], axis=0)
    cx = boxes3d[:, :, 0]
    cy = boxes3d[:, :, 1]
    dz = boxes3d[:, :, 5]
    czc = boxes3d[:, :, 2] + dz / 2.0
    hx = boxes3d[:, :, 3] / 2.0
    hy = boxes3d[:, :, 4] / 2.0
    hz = dz / 2.0
    rz = boxes3d[:, :, 6]
    boxes_prep = jnp.stack(
        [cx, cy, czc, hx, hy, hz, jnp.cos(rz), jnp.sin(rz)],
        axis=-1).reshape(-1)

    mesh = plsc.VectorSubcoreMesh(core_axis_name="c", subcore_axis_name="s")
    sc = pl.kernel(
        _sc_body,
        out_type=(
            jax.ShapeDtypeStruct((B * M * _S, _FP), jnp.float32),
            jax.ShapeDtypeStruct((B * M,), jnp.int32),
        ),
        mesh=mesh,
        compiler_params=pltpu.CompilerParams(
            needs_layout_passes=False, use_tc_tiling_on_sc=False),
        scratch_types=[
            pltpu.VMEM((_N,), jnp.float32),
            pltpu.VMEM((_N,), jnp.float32),
            pltpu.VMEM((_N,), jnp.float32),
            pltpu.VMEM((_NBOX * 8,), jnp.float32),
            pltpu.VMEM((_IDXCAP,), jnp.int32),
            pltpu.VMEM((_S,), jnp.int32),
            pltpu.VMEM((_S, _FP), jnp.float32),
            pltpu.VMEM((_NBOX,), jnp.int32),
            pltpu.SemaphoreType.DMA,
        ],
    )
    pooled_flat, empty_flat = sc(px, py, pz, feats, boxes_prep)
    return pooled_flat[:, :_F].reshape(B, M, _S, _F), empty_flat.reshape(B, M)


# speedup vs baseline: 21.2133x; 1.6428x over previous
import functools

import jax
import jax.numpy as jnp
from jax import lax
from jax.experimental import pallas as pl
from jax.experimental.pallas import tpu as pltpu
from jax.experimental.pallas import tpu_sc as plsc

_B, _N, _C, _M = 8, 16384, 16, 128
_S = 512
_F = 3 + _C
_FP = 32
_NW = 32
_NBOX = (_B * _M) // _NW
_R = _B * _N
_ZROW = _R
_L = 16
_NCHUNK = _N // _L
_IDXCAP = _N + _L


def _sc_body(px_hbm, py_hbm, pz_hbm, feats_hbm, boxes_hbm,
             pooled_hbm, empty_hbm,
             px_v, py_v, pz_v, boxbuf, idxbuf, fidx, rows_v, emptybuf, sem):
    wid = lax.axis_index("s") * 2 + lax.axis_index("c")
    gbase = wid * _NBOX
    b = gbase // _M
    pltpu.sync_copy(px_hbm.at[pl.ds(b * _N, _N)], px_v)
    pltpu.sync_copy(py_hbm.at[pl.ds(b * _N, _N)], py_v)
    pltpu.sync_copy(pz_hbm.at[pl.ds(b * _N, _N)], pz_v)
    pltpu.sync_copy(boxes_hbm.at[pl.ds(gbase * 8, _NBOX * 8)], boxbuf)
    lanes = lax.iota(jnp.int32, _L)
    lane0 = lanes == 0
    badd = b * _N

    def box_body(i, _):
        def param(p):
            return plsc.load_gather(
                boxbuf, [jnp.full((_L,), i * 8 + p, jnp.int32)])

        cx, cy, czc, hx, hy, hz, cosa, sina = [param(p) for p in range(8)]

        def chunk(base, ptr):
            pxv = px_v[pl.ds(base, _L)]
            pyv = py_v[pl.ds(base, _L)]
            pzv = pz_v[pl.ds(base, _L)]
            dxp = pxv - cx
            dyp = pyv - cy
            lx = dxp * cosa + dyp * sina
            ly = dyp * cosa - dxp * sina
            m = ((jnp.abs(pzv - czc) <= hz)
                 & (jnp.abs(lx) <= hx)
                 & (jnp.abs(ly) <= hy))
            tgt = ptr + plsc.cumsum(m.astype(jnp.int32)) - 1
            tgt = jnp.maximum(tgt, 0)
            plsc.store_scatter(idxbuf, [tgt], base + lanes, mask=m)
            return ptr + plsc.all_reduce_population_count(m)

        ptr = plsc.parallel_loop(
            0, _N, step=_L, unroll=8,
            carry=jnp.zeros((_L,), jnp.int32))(chunk)
        cnt = jnp.minimum(jnp.max(ptr), _S)
        cnt_v = jnp.full((_L,), cnt, jnp.int32)
        empty_v = cnt_v == 0
        cnt_safe = jnp.maximum(cnt_v, 1)

        def build(j0):
            jv = j0 + lanes
            w = lax.rem(jv, cnt_safe)
            src = plsc.load_gather(idxbuf, [w]) + badd
            fidx[pl.ds(j0, _L)] = jnp.where(empty_v, _ZROW, src)

        plsc.parallel_loop(0, _S, step=_L, unroll=4)(build)

        copies = [
            pltpu.async_copy(
                feats_hbm.at[fidx.at[pl.ds(kk * 128, 128)]],
                rows_v.at[pl.ds(kk * 128, 128)], sem)
            for kk in range(_S // 128)
        ]
        for cpy in copies:
            cpy.wait()
        pltpu.sync_copy(rows_v, pooled_hbm.at[pl.ds((gbase + i) * _S, _S)])
        plsc.store_scatter(emptybuf, [jnp.full((_L,), i, jnp.int32)],
                           empty_v.astype(jnp.int32), mask=lane0)
        return 0

    lax.fori_loop(0, _NBOX, box_body, 0)
    pltpu.sync_copy(emptybuf, empty_hbm.at[pl.ds(gbase, _NBOX)])


def kernel(points, point_features, boxes3d):
    B, N, _ = points.shape
    M = boxes3d.shape[1]
    px = points[:, :, 0].reshape(-1)
    py = points[:, :, 1].reshape(-1)
    pz = points[:, :, 2].reshape(-1)
    feats = jnp.concatenate([
        points, point_features,
        jnp.zeros((B, N, _FP - _F), jnp.float32)], axis=-1)
    feats = feats.reshape(B * N, _FP)
    feats = jnp.concatenate(
        [feats, jnp.zeros((_L, _FP), jnp.float32)], axis=0)
    cx = boxes3d[:, :, 0]
    cy = boxes3d[:, :, 1]
    dz = boxes3d[:, :, 5]
    czc = boxes3d[:, :, 2] + dz / 2.0
    hx = boxes3d[:, :, 3] / 2.0
    hy = boxes3d[:, :, 4] / 2.0
    hz = dz / 2.0
    rz = boxes3d[:, :, 6]
    boxes_prep = jnp.stack(
        [cx, cy, czc, hx, hy, hz, jnp.cos(rz), jnp.sin(rz)],
        axis=-1).reshape(-1)

    mesh = plsc.VectorSubcoreMesh(core_axis_name="c", subcore_axis_name="s")
    sc = pl.kernel(
        _sc_body,
        out_type=(
            jax.ShapeDtypeStruct((B * M * _S, _FP), jnp.float32),
            jax.ShapeDtypeStruct((B * M,), jnp.int32),
        ),
        mesh=mesh,
        compiler_params=pltpu.CompilerParams(
            needs_layout_passes=False, use_tc_tiling_on_sc=False),
        scratch_types=[
            pltpu.VMEM((_N,), jnp.float32),
            pltpu.VMEM((_N,), jnp.float32),
            pltpu.VMEM((_N,), jnp.float32),
            pltpu.VMEM((_NBOX * 8,), jnp.float32),
            pltpu.VMEM((_IDXCAP,), jnp.int32),
            pltpu.VMEM((_S,), jnp.int32),
            pltpu.VMEM((_S, _FP), jnp.float32),
            pltpu.VMEM((_NBOX,), jnp.int32),
            pltpu.SemaphoreType.DMA,
        ],
    )
    pooled_flat, empty_flat = sc(px, py, pz, feats, boxes_prep)
    return pooled_flat[:, :_F].reshape(B, M, _S, _F), empty_flat.reshape(B, M)


# speedup vs baseline: 22.5770x; 1.0643x over previous
import functools

import jax
import jax.numpy as jnp
from jax import lax
from jax.experimental import pallas as pl
from jax.experimental.pallas import tpu as pltpu
from jax.experimental.pallas import tpu_sc as plsc

_B, _N, _C, _M = 8, 16384, 16, 128
_S = 512
_F = 3 + _C
_FP = 32
_NW = 32
_NBOX = (_B * _M) // _NW
_R = _B * _N
_ZROW = _R
_L = 16
_NCHUNK = _N // _L
_IDXCAP = _N + _L


def _sc_body(px_hbm, py_hbm, pz_hbm, feats_hbm, boxes_hbm,
             pooled_hbm, empty_hbm,
             px_v, py_v, pz_v, boxbuf, idxbuf, fidx, rows_v, emptybuf, sem):
    wid = lax.axis_index("s") * 2 + lax.axis_index("c")
    gbase = wid * _NBOX
    b = gbase // _M
    pltpu.sync_copy(px_hbm.at[pl.ds(b * _N, _N)], px_v)
    pltpu.sync_copy(py_hbm.at[pl.ds(b * _N, _N)], py_v)
    pltpu.sync_copy(pz_hbm.at[pl.ds(b * _N, _N)], pz_v)
    pltpu.sync_copy(boxes_hbm.at[pl.ds(gbase * 8, _NBOX * 8)], boxbuf)
    lanes = lax.iota(jnp.int32, _L)
    lane0 = lanes == 0
    badd = b * _N

    def box_body(i, _):
        def param(p):
            return plsc.load_gather(
                boxbuf, [jnp.full((_L,), i * 8 + p, jnp.int32)])

        cx, cy, czc, hx, hy, hz, cosa, sina = [param(p) for p in range(8)]

        def chunk(base, ptr):
            pxv = px_v[pl.ds(base, _L)]
            pyv = py_v[pl.ds(base, _L)]
            pzv = pz_v[pl.ds(base, _L)]
            dxp = pxv - cx
            dyp = pyv - cy
            lx = dxp * cosa + dyp * sina
            ly = dyp * cosa - dxp * sina
            m = ((jnp.abs(pzv - czc) <= hz)
                 & (jnp.abs(lx) <= hx)
                 & (jnp.abs(ly) <= hy))
            plsc.store_compressed(
                idxbuf.at[pl.ds(ptr, _L)], base + lanes, mask=m)
            return ptr + plsc.all_reduce_population_count(m)[0]

        ptr = plsc.parallel_loop(
            0, _N, step=_L, unroll=8, carry=jnp.int32(0))(chunk)
        cnt = jnp.minimum(ptr, _S)
        cnt_v = jnp.full((_L,), cnt, jnp.int32)
        empty_v = cnt_v == 0
        cnt_safe = jnp.maximum(cnt_v, 1)

        def build(j0):
            jv = j0 + lanes
            w = lax.rem(jv, cnt_safe)
            src = plsc.load_gather(idxbuf, [w]) + badd
            fidx[pl.ds(j0, _L)] = jnp.where(empty_v, _ZROW, src)

        plsc.parallel_loop(0, _S, step=_L, unroll=4)(build)

        copies = [
            pltpu.async_copy(
                feats_hbm.at[fidx.at[pl.ds(kk * 128, 128)]],
                rows_v.at[pl.ds(kk * 128, 128)], sem)
            for kk in range(_S // 128)
        ]
        for cpy in copies:
            cpy.wait()
        pltpu.sync_copy(rows_v, pooled_hbm.at[pl.ds((gbase + i) * _S, _S)])
        plsc.store_scatter(emptybuf, [jnp.full((_L,), i, jnp.int32)],
                           empty_v.astype(jnp.int32), mask=lane0)
        return 0

    lax.fori_loop(0, _NBOX, box_body, 0)
    pltpu.sync_copy(emptybuf, empty_hbm.at[pl.ds(gbase, _NBOX)])


def kernel(points, point_features, boxes3d):
    B, N, _ = points.shape
    M = boxes3d.shape[1]
    px = points[:, :, 0].reshape(-1)
    py = points[:, :, 1].reshape(-1)
    pz = points[:, :, 2].reshape(-1)
    feats = jnp.concatenate([
        points, point_features,
        jnp.zeros((B, N, _FP - _F), jnp.float32)], axis=-1)
    feats = feats.reshape(B * N, _FP)
    feats = jnp.concatenate(
        [feats, jnp.zeros((_L, _FP), jnp.float32)], axis=0)
    cx = boxes3d[:, :, 0]
    cy = boxes3d[:, :, 1]
    dz = boxes3d[:, :, 5]
    czc = boxes3d[:, :, 2] + dz / 2.0
    hx = boxes3d[:, :, 3] / 2.0
    hy = boxes3d[:, :, 4] / 2.0
    hz = dz / 2.0
    rz = boxes3d[:, :, 6]
    boxes_prep = jnp.stack(
        [cx, cy, czc, hx, hy, hz, jnp.cos(rz), jnp.sin(rz)],
        axis=-1).reshape(-1)

    mesh = plsc.VectorSubcoreMesh(core_axis_name="c", subcore_axis_name="s")
    sc = pl.kernel(
        _sc_body,
        out_type=(
            jax.ShapeDtypeStruct((B * M * _S, _FP), jnp.float32),
            jax.ShapeDtypeStruct((B * M,), jnp.int32),
        ),
        mesh=mesh,
        compiler_params=pltpu.CompilerParams(
            needs_layout_passes=False, use_tc_tiling_on_sc=False),
        scratch_types=[
            pltpu.VMEM((_N,), jnp.float32),
            pltpu.VMEM((_N,), jnp.float32),
            pltpu.VMEM((_N,), jnp.float32),
            pltpu.VMEM((_NBOX * 8,), jnp.float32),
            pltpu.VMEM((_IDXCAP,), jnp.int32),
            pltpu.VMEM((_S,), jnp.int32),
            pltpu.VMEM((_S, _FP), jnp.float32),
            pltpu.VMEM((_NBOX,), jnp.int32),
            pltpu.SemaphoreType.DMA,
        ],
    )
    pooled_flat, empty_flat = sc(px, py, pz, feats, boxes_prep)
    return pooled_flat[:, :_F].reshape(B, M, _S, _F), empty_flat.reshape(B, M)


# speedup vs baseline: 23.7302x; 1.0511x over previous
import functools

import jax
import jax.numpy as jnp
from jax import lax
from jax.experimental import pallas as pl
from jax.experimental.pallas import tpu as pltpu
from jax.experimental.pallas import tpu_sc as plsc

_B, _N, _C, _M = 8, 16384, 16, 128
_S = 512
_F = 3 + _C
_FP = 32
_NW = 32
_NBOX = (_B * _M) // _NW
_R = _B * _N
_ZROW = _R
_L = 16
_NCHUNK = _N // _L
_IDXCAP = _N + _L


def _sc_body(px_hbm, py_hbm, pz_hbm, feats_hbm, boxes_hbm,
             feat_out_hbm, xyz_out_hbm, empty_hbm,
             px_v, py_v, pz_v, boxbuf, idxbuf, fidx, rows_v, xyzbuf,
             emptybuf, sem):
    wid = lax.axis_index("s") * 2 + lax.axis_index("c")
    gbase = wid * _NBOX
    b = gbase // _M
    pltpu.sync_copy(px_hbm.at[pl.ds(b * _N, _N)], px_v)
    pltpu.sync_copy(py_hbm.at[pl.ds(b * _N, _N)], py_v)
    pltpu.sync_copy(pz_hbm.at[pl.ds(b * _N, _N)], pz_v)
    pltpu.sync_copy(boxes_hbm.at[pl.ds(gbase * 8, _NBOX * 8)], boxbuf)
    lanes = lax.iota(jnp.int32, _L)
    lane0 = lanes == 0
    badd = b * _N

    def box_body(i, _):
        def param(p):
            return plsc.load_gather(
                boxbuf, [jnp.full((_L,), i * 8 + p, jnp.int32)])

        cx, cy, czc, hx, hy, hz, cosa, sina = [param(p) for p in range(8)]

        def chunk(base, ptr):
            pxv = px_v[pl.ds(base, _L)]
            pyv = py_v[pl.ds(base, _L)]
            pzv = pz_v[pl.ds(base, _L)]
            dxp = pxv - cx
            dyp = pyv - cy
            lx = dxp * cosa + dyp * sina
            ly = dyp * cosa - dxp * sina
            m = ((jnp.abs(pzv - czc) <= hz)
                 & (jnp.abs(lx) <= hx)
                 & (jnp.abs(ly) <= hy))
            plsc.store_compressed(
                idxbuf.at[pl.ds(ptr, _L)], base + lanes, mask=m)
            return ptr + plsc.all_reduce_population_count(m)[0]

        ptr = plsc.parallel_loop(
            0, _N, step=_L, unroll=8, carry=jnp.int32(0))(chunk)
        cnt = jnp.minimum(ptr, _S)
        cnt_v = jnp.full((_L,), cnt, jnp.int32)
        empty_v = cnt_v == 0
        cnt_safe = jnp.maximum(cnt_v, 1)
        nzf = jnp.where(empty_v, 0.0, 1.0)

        def build(j0):
            jv = j0 + lanes
            w = lax.rem(jv, cnt_safe)
            src = plsc.load_gather(idxbuf, [w])
            src = jnp.where(empty_v, 0, src)
            fidx[pl.ds(j0, _L)] = jnp.where(empty_v, _ZROW, src + badd)
            xyzbuf[pl.ds(j0, _L)] = plsc.load_gather(px_v, [src]) * nzf
            xyzbuf[pl.ds(_S + j0, _L)] = plsc.load_gather(py_v, [src]) * nzf
            xyzbuf[pl.ds(2 * _S + j0, _L)] = plsc.load_gather(pz_v, [src]) * nzf

        plsc.parallel_loop(0, _S, step=_L, unroll=4)(build)

        copies = [
            pltpu.async_copy(
                feats_hbm.at[fidx.at[pl.ds(kk * 128, 128)]],
                rows_v.at[pl.ds(kk * 128, 128)], sem)
            for kk in range(_S // 128)
        ]
        for cpy in copies:
            cpy.wait()
        pltpu.sync_copy(rows_v, feat_out_hbm.at[pl.ds((gbase + i) * _S, _S)])
        pltpu.sync_copy(
            xyzbuf, xyz_out_hbm.at[pl.ds((gbase + i) * 3 * _S, 3 * _S)])
        plsc.store_scatter(emptybuf, [jnp.full((_L,), i, jnp.int32)],
                           empty_v.astype(jnp.int32), mask=lane0)
        return 0

    lax.fori_loop(0, _NBOX, box_body, 0)
    pltpu.sync_copy(emptybuf, empty_hbm.at[pl.ds(gbase, _NBOX)])


def kernel(points, point_features, boxes3d):
    B, N, _ = points.shape
    M = boxes3d.shape[1]
    px = points[:, :, 0].reshape(-1)
    py = points[:, :, 1].reshape(-1)
    pz = points[:, :, 2].reshape(-1)
    feats = jnp.concatenate(
        [point_features.reshape(B * N, _C),
         jnp.zeros((_L, _C), jnp.float32)], axis=0)
    cx = boxes3d[:, :, 0]
    cy = boxes3d[:, :, 1]
    dz = boxes3d[:, :, 5]
    czc = boxes3d[:, :, 2] + dz / 2.0
    hx = boxes3d[:, :, 3] / 2.0
    hy = boxes3d[:, :, 4] / 2.0
    hz = dz / 2.0
    rz = boxes3d[:, :, 6]
    boxes_prep = jnp.stack(
        [cx, cy, czc, hx, hy, hz, jnp.cos(rz), jnp.sin(rz)],
        axis=-1).reshape(-1)

    mesh = plsc.VectorSubcoreMesh(core_axis_name="c", subcore_axis_name="s")
    sc = pl.kernel(
        _sc_body,
        out_type=(
            jax.ShapeDtypeStruct((B * M * _S, _C), jnp.float32),
            jax.ShapeDtypeStruct((B * M * 3 * _S,), jnp.float32),
            jax.ShapeDtypeStruct((B * M,), jnp.int32),
        ),
        mesh=mesh,
        compiler_params=pltpu.CompilerParams(
            needs_layout_passes=False, use_tc_tiling_on_sc=False),
        scratch_types=[
            pltpu.VMEM((_N,), jnp.float32),
            pltpu.VMEM((_N,), jnp.float32),
            pltpu.VMEM((_N,), jnp.float32),
            pltpu.VMEM((_NBOX * 8,), jnp.float32),
            pltpu.VMEM((_IDXCAP,), jnp.int32),
            pltpu.VMEM((_S,), jnp.int32),
            pltpu.VMEM((_S, _C), jnp.float32),
            pltpu.VMEM((3 * _S,), jnp.float32),
            pltpu.VMEM((_NBOX,), jnp.int32),
            pltpu.SemaphoreType.DMA,
        ],
    )
    feat_out, xyz_out, empty_flat = sc(px, py, pz, feats, boxes_prep)
    xyz = xyz_out.reshape(B, M, 3, _S).transpose(0, 1, 3, 2)
    pooled = jnp.concatenate([xyz, feat_out.reshape(B, M, _S, _C)], axis=-1)
    return pooled, empty_flat.reshape(B, M)
